# Initial kernel scaffold; baseline (speedup 1.0000x reference)
import functools

import jax
import jax.numpy as jnp
from jax import lax
from jax.experimental import pallas as pl
from jax.experimental.pallas import tpu as pltpu
from jax.experimental.pallas import tpu_sc as plsc

NEG_SLOPE = 0.2
_NC, _NS, _L = 2, 16, 16
_CH = 80


def _proj_body(x_ref, ws_ref, bs_ref, wl_ref, bl_ref, wr_ref, br_ref,
               fs_ref, el_ref, er_ref):
    x = x_ref[...]
    hi = jax.lax.Precision.HIGHEST
    fs_ref[...] = jnp.dot(x, ws_ref[...], precision=hi,
                          preferred_element_type=jnp.float32) + bs_ref[...]
    el_ref[...] = jnp.dot(x, wl_ref[...], precision=hi,
                          preferred_element_type=jnp.float32) + bl_ref[...]
    er_ref[...] = jnp.dot(x, wr_ref[...], precision=hi,
                          preferred_element_type=jnp.float32) + br_ref[...]


def _edge_body(n, k, d_out, ew, nchunk,
               fs_hbm, el_hbm, er_hbm, src_hbm, dst_hbm, z_acc_hbm, z_den_hbm,
               acc_out, den_out,
               acc, den, src_v, dst_v, l_v, r_v, s_v, f_v, sem0, sem1, sem2):
    cid = lax.axis_index("c")
    sid = lax.axis_index("s")
    wid = sid * _NC + cid

    rows = n // _NS
    r0 = sid * rows
    pltpu.sync_copy(z_acc_hbm.at[pl.ds(r0, rows)], acc.at[pl.ds(r0, rows)])
    pltpu.sync_copy(z_den_hbm.at[pl.ds(r0, rows)], den.at[pl.ds(r0, rows)])
    plsc.subcore_barrier()

    def chunk_body(c, carry):
        off = wid * ew + c * _CH
        pltpu.sync_copy(src_hbm.at[pl.ds(off, _CH)], src_v)
        pltpu.sync_copy(dst_hbm.at[pl.ds(off, _CH)], dst_v)
        gl = pltpu.async_copy(el_hbm.at[src_v], l_v, sem0)
        gr = pltpu.async_copy(er_hbm.at[dst_v], r_v, sem1)
        gf = pltpu.async_copy(fs_hbm.at[src_v], f_v, sem2)
        gl.wait()
        gr.wait()

        def s_body(i, carry2):
            v = l_v[i, :] + r_v[i, :]
            v = jnp.where(v > 0.0, v, NEG_SLOPE * v)
            s_v[i, :] = jnp.exp(v)
            return carry2

        lax.fori_loop(0, _CH, s_body, 0)
        pltpu.sync_copy(s_v, den.at[dst_v], add=True)
        gf.wait()

        def m_body(i, carry2):
            for kk in range(k):
                sl = pl.ds(kk * d_out, d_out)
                f_v[i, sl] = f_v[i, sl] * s_v[i, kk]
            return carry2

        lax.fori_loop(0, _CH, m_body, 0)
        pltpu.sync_copy(f_v, acc.at[dst_v], add=True)
        return carry

    lax.fori_loop(0, nchunk, chunk_body, 0)
    plsc.subcore_barrier()

    pltpu.sync_copy(acc.at[pl.ds(r0, rows)], acc_out.at[cid, pl.ds(r0, rows)])
    pltpu.sync_copy(den.at[pl.ds(r0, rows)], den_out.at[cid, pl.ds(r0, rows)])


def _combine_body(acc_ref, den_ref, em_ref, o_ref):
    a = acc_ref[0] + acc_ref[1]
    d = den_ref[0] + den_ref[1]
    dexp = jnp.dot(d, em_ref[...], precision=jax.lax.Precision.HIGHEST,
                   preferred_element_type=jnp.float32)
    o_ref[...] = jnp.where(dexp > 0.0, a / dexp, 0.0)


def kernel(feat, edge_index, W_src, b_src, W_dst, b_dst, attn_src):
    n, d_in = feat.shape
    e = edge_index.shape[1]
    k = attn_src.shape[0]
    d_out = attn_src.shape[1] // 2
    kd = k * d_out
    f32 = jnp.float32

    nw = _NC * _NS
    assert e % (nw * _CH) == 0 and n % _NS == 0 and d_out == _L
    ew = e // nw
    nchunk = ew // _CH

    attn_l = attn_src[:, :d_out]
    attn_r = attn_src[:, d_out:]
    w_el = jnp.einsum('dkc,kc->dk', W_src.reshape(d_in, k, d_out), attn_l)
    b_el = jnp.einsum('kc,kc->k', b_src.reshape(k, d_out), attn_l)
    w_er = jnp.einsum('dkc,kc->dk', W_dst.reshape(d_in, k, d_out), attn_r)
    b_er = jnp.einsum('kc,kc->k', b_dst.reshape(k, d_out), attn_r)
    pad = _L - k
    w_el_p = jnp.pad(w_el, ((0, 0), (0, pad)))
    w_er_p = jnp.pad(w_er, ((0, 0), (0, pad)))
    b_el_p = jnp.pad(b_el, ((0, pad),))
    b_er_p = jnp.pad(b_er, ((0, pad),))

    fs, elp, erp = pl.pallas_call(
        _proj_body,
        out_shape=(
            jax.ShapeDtypeStruct((n, kd), f32),
            jax.ShapeDtypeStruct((n, _L), f32),
            jax.ShapeDtypeStruct((n, _L), f32),
        ),
    )(feat, W_src, b_src, w_el_p, b_el_p, w_er_p, b_er_p)

    src = edge_index[0]
    dst = edge_index[1]
    z_acc = jnp.zeros((n, kd), f32)
    z_den = jnp.zeros((n, _L), f32)

    sc_edge = pl.kernel(
        functools.partial(_edge_body, n, k, d_out, ew, nchunk),
        out_type=(
            jax.ShapeDtypeStruct((_NC, n, kd), f32),
            jax.ShapeDtypeStruct((_NC, n, _L), f32),
        ),
        mesh=plsc.VectorSubcoreMesh(core_axis_name="c", subcore_axis_name="s"),
        scratch_types=[
            pltpu.VMEM_SHARED((n, kd), f32),
            pltpu.VMEM_SHARED((n, _L), f32),
            pltpu.VMEM((_CH,), jnp.int32),
            pltpu.VMEM((_CH,), jnp.int32),
            pltpu.VMEM((_CH, _L), f32),
            pltpu.VMEM((_CH, _L), f32),
            pltpu.VMEM((_CH, _L), f32),
            pltpu.VMEM((_CH, kd), f32),
            pltpu.SemaphoreType.DMA,
            pltpu.SemaphoreType.DMA,
            pltpu.SemaphoreType.DMA,
        ],
    )
    acc_p, den_p = sc_edge(fs, elp, erp, src, dst, z_acc, z_den)

    emat = (jnp.arange(kd)[None, :] // d_out ==
            jnp.arange(_L)[:, None]).astype(f32)
    out = pl.pallas_call(
        _combine_body,
        out_shape=jax.ShapeDtypeStruct((n, kd), f32),
    )(acc_p, den_p, emat)
    return out


# speedup vs baseline: 78.5061x; 78.5061x over previous
import functools

import jax
import jax.numpy as jnp
from jax import lax
from jax.experimental import pallas as pl
from jax.experimental.pallas import tpu as pltpu
from jax.experimental.pallas import tpu_sc as plsc

NEG_SLOPE = 0.2
_NC, _NS, _L = 2, 16, 16
_CH = 80


def _proj_body(x_ref, ws_ref, bs_ref, wl_ref, bl_ref, wr_ref, br_ref,
               fs_ref, el_ref, er_ref):
    x = x_ref[...]
    hi = jax.lax.Precision.HIGHEST
    fs_ref[...] = jnp.dot(x, ws_ref[...], precision=hi,
                          preferred_element_type=jnp.float32) + bs_ref[...]
    el_ref[...] = jnp.dot(x, wl_ref[...], precision=hi,
                          preferred_element_type=jnp.float32) + bl_ref[...]
    er_ref[...] = jnp.dot(x, wr_ref[...], precision=hi,
                          preferred_element_type=jnp.float32) + br_ref[...]


def _edge_body(n_pad, k, d_out, ew, nchunk,
               fs_hbm, el_hbm, er_hbm, src_hbm, dst_hbm, z_acc_hbm, z_den_hbm,
               acc_out, den_out,
               acc, den, src_v, dst_v, l_v, r_v, s_v, f_v, sem0, sem1, sem2):
    cid = lax.axis_index("c")
    sid = lax.axis_index("s")
    wid = sid * _NC + cid

    rows = n_pad // _NS
    r0 = sid * rows
    pltpu.sync_copy(z_acc_hbm.at[pl.ds(r0, rows)], acc.at[pl.ds(r0, rows)])
    pltpu.sync_copy(z_den_hbm.at[pl.ds(r0, rows)], den.at[pl.ds(r0, rows)])
    plsc.subcore_barrier()

    def chunk_body(c, carry):
        off = wid * ew + c * _CH
        pltpu.sync_copy(src_hbm.at[pl.ds(off, _CH)], src_v)
        pltpu.sync_copy(dst_hbm.at[pl.ds(off, _CH)], dst_v)
        gl = pltpu.async_copy(el_hbm.at[src_v], l_v, sem0)
        gr = pltpu.async_copy(er_hbm.at[dst_v], r_v, sem1)
        gf = pltpu.async_copy(fs_hbm.at[src_v], f_v, sem2)
        gl.wait()
        gr.wait()

        def s_body(i, carry2):
            v = l_v[i, :] + r_v[i, :]
            v = jnp.where(v > 0.0, v, NEG_SLOPE * v)
            s_v[i, :] = jnp.exp(v)
            return carry2

        lax.fori_loop(0, _CH, s_body, 0)
        pltpu.sync_copy(s_v, den.at[dst_v], add=True)
        gf.wait()

        def m_body(i, carry2):
            s_vec = s_v[i, :]
            for kk in range(k):
                sl = pl.ds(kk * d_out, d_out)
                f_v[i, sl] = f_v[i, sl] * s_vec[kk]
            return carry2

        lax.fori_loop(0, _CH, m_body, 0)
        pltpu.sync_copy(f_v, acc.at[dst_v], add=True)
        return carry

    lax.fori_loop(0, nchunk, chunk_body, 0)
    plsc.subcore_barrier()

    pltpu.sync_copy(acc.at[pl.ds(r0, rows)], acc_out.at[cid, pl.ds(r0, rows)])
    pltpu.sync_copy(den.at[pl.ds(r0, rows)], den_out.at[cid, pl.ds(r0, rows)])


def _combine_body(acc_ref, den_ref, em_ref, o_ref):
    a = acc_ref[0] + acc_ref[1]
    d = den_ref[0] + den_ref[1]
    dexp = jnp.dot(d, em_ref[...], precision=jax.lax.Precision.HIGHEST,
                   preferred_element_type=jnp.float32)
    o_ref[...] = jnp.where(dexp > 0.0, a / dexp, 0.0)


def kernel(feat, edge_index, W_src, b_src, W_dst, b_dst, attn_src):
    n, d_in = feat.shape
    e = edge_index.shape[1]
    k = attn_src.shape[0]
    d_out = attn_src.shape[1] // 2
    kd = k * d_out
    f32 = jnp.float32

    nw = _NC * _NS
    assert e % (nw * _CH) == 0 and d_out == _L
    ew = e // nw
    nchunk = ew // _CH
    n_pad = ((n + 8 * _NS - 1) // (8 * _NS)) * (8 * _NS)

    attn_l = attn_src[:, :d_out]
    attn_r = attn_src[:, d_out:]
    w_el = jnp.einsum('dkc,kc->dk', W_src.reshape(d_in, k, d_out), attn_l)
    b_el = jnp.einsum('kc,kc->k', b_src.reshape(k, d_out), attn_l)
    w_er = jnp.einsum('dkc,kc->dk', W_dst.reshape(d_in, k, d_out), attn_r)
    b_er = jnp.einsum('kc,kc->k', b_dst.reshape(k, d_out), attn_r)
    pad = _L - k
    w_el_p = jnp.pad(w_el, ((0, 0), (0, pad)))
    w_er_p = jnp.pad(w_er, ((0, 0), (0, pad)))
    b_el_p = jnp.pad(b_el, ((0, pad),))
    b_er_p = jnp.pad(b_er, ((0, pad),))

    fs, elp, erp = pl.pallas_call(
        _proj_body,
        out_shape=(
            jax.ShapeDtypeStruct((n, kd), f32),
            jax.ShapeDtypeStruct((n, _L), f32),
            jax.ShapeDtypeStruct((n, _L), f32),
        ),
    )(feat, W_src, b_src, w_el_p, b_el_p, w_er_p, b_er_p)

    src = edge_index[0]
    dst = edge_index[1]
    z_acc = jnp.zeros((n_pad, kd), f32)
    z_den = jnp.zeros((n_pad, _L), f32)

    sc_edge = pl.kernel(
        functools.partial(_edge_body, n_pad, k, d_out, ew, nchunk),
        out_type=(
            jax.ShapeDtypeStruct((_NC, n_pad, kd), f32),
            jax.ShapeDtypeStruct((_NC, n_pad, _L), f32),
        ),
        mesh=plsc.VectorSubcoreMesh(core_axis_name="c", subcore_axis_name="s"),
        scratch_types=[
            pltpu.VMEM_SHARED((n_pad, kd), f32),
            pltpu.VMEM_SHARED((n_pad, _L), f32),
            pltpu.VMEM((_CH,), jnp.int32),
            pltpu.VMEM((_CH,), jnp.int32),
            pltpu.VMEM((_CH, _L), f32),
            pltpu.VMEM((_CH, _L), f32),
            pltpu.VMEM((_CH, _L), f32),
            pltpu.VMEM((_CH, kd), f32),
            pltpu.SemaphoreType.DMA,
            pltpu.SemaphoreType.DMA,
            pltpu.SemaphoreType.DMA,
        ],
        compiler_params=pltpu.CompilerParams(use_tc_tiling_on_sc=False),
    )
    acc_p, den_p = sc_edge(fs, elp, erp, src, dst, z_acc, z_den)

    emat = (jnp.arange(kd)[None, :] // d_out ==
            jnp.arange(_L)[:, None]).astype(f32)
    out = pl.pallas_call(
        _combine_body,
        out_shape=jax.ShapeDtypeStruct((n_pad, kd), f32),
    )(acc_p, den_p, emat)
    return out[:n]
